# baseline (speedup 1.0000x reference)
import functools

import jax
import jax.numpy as jnp
from jax.experimental import pallas as pl


def _avg_pool3(x):
    s = jax.lax.reduce_window(x, 0.0, jax.lax.add, (1, 1, 3, 3), (1, 1, 1, 1),
                              ((0, 0), (0, 0), (1, 1), (1, 1)))
    return s / 9.0


def _fused_body(x0_ref, x1_ref, x2_ref, w0_ref, w1_ref, w2_ref, ct_ref,
                cent_ref, centers_ref, out_ref):
    e = (jnp.dot(w0_ref[...], x0_ref[0], preferred_element_type=jnp.float32)
         + jnp.dot(w1_ref[...], x1_ref[0], preferred_element_type=jnp.float32)
         + jnp.dot(w2_ref[...], x2_ref[0], preferred_element_type=jnp.float32)
         + ct_ref[...])
    feats = jnp.sum(e * e, axis=0)[:, None]
    eb = e.astype(jnp.bfloat16)
    prod = jax.lax.dot_general(eb, cent_ref[...],
                               (((0,), (0,)), ((), ())),
                               preferred_element_type=jnp.float32)
    d2 = feats + centers_ref[...] - 2.0 * prod

    iota = jax.lax.broadcasted_iota(jnp.int32, d2.shape, 1)
    cur = d2
    mins = []
    for _ in range(3):
        mins.append(jnp.min(cur, axis=1))
        am = jnp.argmin(cur, axis=1)
        cur = jnp.where(iota == am[:, None], jnp.inf, cur)
    d0 = jnp.sqrt(jnp.maximum(mins[0], 1e-12))
    d1 = jnp.sqrt(jnp.maximum(mins[1], 1e-12))
    d2s = jnp.sqrt(jnp.maximum(mins[2], 1e-12))
    sm0 = 1.0 / (1.0 + jnp.exp(d0 - d1) + jnp.exp(d0 - d2s))
    out_ref[0, 0] = sm0 * d0


@functools.partial(jax.jit, static_argnums=())
def kernel(p0, p1, p2, W, bconv, centroids):
    b = p0.shape[0]
    h, w = p0.shape[2], p0.shape[3]
    hw = h * w
    c = centroids.shape[0]
    n = centroids.shape[1]
    c0, c1, c2 = p0.shape[1], p1.shape[1], p2.shape[1]

    bm = 512
    hw_pad = -(-hw // bm) * bm
    pad = hw_pad - hw

    def prep(p):
        a = _avg_pool3(p)
        if a.shape[2] != h:
            a = jax.image.resize(a, (b, a.shape[1], h, w), method='bilinear')
        a = a.reshape(b, a.shape[1], hw).astype(jnp.bfloat16)
        return jnp.pad(a, ((0, 0), (0, 0), (0, pad)))

    x0, x1, x2 = prep(p0), prep(p1), prep(p2)

    xx = (jnp.arange(w, dtype=jnp.float32) / (w - 1)) * 2.0 - 1.0
    yy = (jnp.arange(h, dtype=jnp.float32) / (h - 1)) * 2.0 - 1.0
    grid_x = jnp.pad(jnp.tile(xx, h), (0, pad))
    grid_y = jnp.pad(jnp.repeat(yy, w), (0, pad))
    ct = (W[:, c, None] * grid_x[None, :] + W[:, c + 1, None] * grid_y[None, :]
          + bconv[:, None])
    w0 = W[:, :c0].astype(jnp.bfloat16)
    w1 = W[:, c0:c0 + c1].astype(jnp.bfloat16)
    w2 = W[:, c0 + c1:c].astype(jnp.bfloat16)
    centb = centroids.astype(jnp.bfloat16)
    centers = jnp.sum(centroids * centroids, axis=0, keepdims=True)

    nblk = hw_pad // bm
    score = pl.pallas_call(
        _fused_body,
        grid=(b, nblk),
        in_specs=[
            pl.BlockSpec((1, c0, bm), lambda i, j: (i, 0, j)),
            pl.BlockSpec((1, c1, bm), lambda i, j: (i, 0, j)),
            pl.BlockSpec((1, c2, bm), lambda i, j: (i, 0, j)),
            pl.BlockSpec((c, c0), lambda i, j: (0, 0)),
            pl.BlockSpec((c, c1), lambda i, j: (0, 0)),
            pl.BlockSpec((c, c2), lambda i, j: (0, 0)),
            pl.BlockSpec((c, bm), lambda i, j: (0, j)),
            pl.BlockSpec((c, n), lambda i, j: (0, 0)),
            pl.BlockSpec((1, n), lambda i, j: (0, 0)),
        ],
        out_specs=pl.BlockSpec((1, 1, bm), lambda i, j: (i * nblk + j, 0, 0)),
        out_shape=jax.ShapeDtypeStruct((b * nblk, 1, bm), jnp.float32),
    )(x0, x1, x2, w0, w1, w2, ct, centb, centers)

    return score.reshape(b, hw_pad)[:, :hw].reshape(b, 1, h, w)
